# baseline (speedup 1.0000x reference)
import functools

import jax
import jax.numpy as jnp
from jax import lax
from jax.experimental import pallas as pl
from jax.experimental.pallas import tpu as pltpu
from jax.experimental.pallas import tpu_sc as plsc

N, E, D, C = 10000, 160000, 256, 64
HALF = D // 2
NC, NS = 2, 16
CHUNK = 128
HC = 40
CH = 2 * HC
EP = NS * CH * CHUNK
NP = 10240
STRIPE = NP // NS
LANES = 16
MASK14 = (1 << 14) - 1


def _fill(buf, val):
  vec = jnp.full((LANES,), val, jnp.float32)

  @pl.loop(0, CHUNK)
  def _(r):
    @pl.loop(0, HALF, step=LANES)
    def _(j):
      buf[r, pl.ds(j, LANES)] = vec


def _sc_body(x_hbm, epk, agg_out, cnt_out,
             pkh, idx_dst, rows0, rows1, agg_sh, sem0, sem1, sem2, sem3):
  c = lax.axis_index("c")
  s = lax.axis_index("s")
  base = s * STRIPE
  obase = c * NP + base
  xcol = x_hbm.at[:, pl.ds(pl.multiple_of(c * HALF, HALF), HALF)]

  _fill(rows0, 0.0)
  for j in range(STRIPE // CHUNK):
    pltpu.sync_copy(rows0, agg_sh.at[pl.ds(base + j * CHUNK, CHUNK)])

  plsc.subcore_barrier()

  for h in range(CH // HC):
    pltpu.sync_copy(epk.at[pl.ds(s * CH + h * HC, HC)], pkh)

    @pl.loop(0, HC)
    def _(r):
      @pl.loop(0, CHUNK, step=LANES)
      def _(j):
        v = pkh[r, pl.ds(j, LANES)]
        idx_dst[h * HC + r, pl.ds(j, LANES)] = v >> 14
        pkh[r, pl.ds(j, LANES)] = v & MASK14

    pltpu.make_async_copy(xcol.at[pkh.at[0]], rows0, sem0).start()
    pltpu.make_async_copy(xcol.at[pkh.at[1]], rows1, sem1).start()

    @pl.loop(0, HC, step=2)
    def _(g):
      pltpu.make_async_copy(xcol.at[pkh.at[g]], rows0, sem0).wait()
      pltpu.sync_copy(rows0, agg_sh.at[idx_dst.at[h * HC + g]], add=True)

      @pl.when(g + 2 < HC)
      def _():
        pltpu.make_async_copy(xcol.at[pkh.at[g + 2]], rows0, sem0).start()

      pltpu.make_async_copy(xcol.at[pkh.at[g + 1]], rows1, sem1).wait()
      pltpu.sync_copy(rows1, agg_sh.at[idx_dst.at[h * HC + g + 1]],
                      add=True)

      @pl.when(g + 3 < HC)
      def _():
        pltpu.make_async_copy(xcol.at[pkh.at[g + 3]], rows1, sem1).start()

  plsc.subcore_barrier()

  pltpu.sync_copy(agg_sh.at[pl.ds(base, STRIPE)],
                  agg_out.at[pl.ds(obase, STRIPE)])
  _fill(rows0, 0.0)
  for j in range(STRIPE // CHUNK):
    pltpu.sync_copy(rows0, agg_sh.at[pl.ds(base + j * CHUNK, CHUNK)])

  plsc.subcore_barrier()

  _fill(rows1, 1.0)

  @pl.loop(0, CH, step=8)
  def _(g):
    for k in range(4):
      pltpu.make_async_copy(
          rows1, agg_sh.at[idx_dst.at[g + 2 * k + c]], sem2).start(add=True)
    for k in range(4):
      pltpu.make_async_copy(
          rows1, agg_sh.at[idx_dst.at[g + 2 * k + c]], sem2).wait()

  plsc.subcore_barrier()

  pltpu.sync_copy(agg_sh.at[pl.ds(base, STRIPE)],
                  cnt_out.at[pl.ds(obase, STRIPE)])


def _sc_aggregate(x, epk):
  mesh = plsc.VectorSubcoreMesh(core_axis_name="c", subcore_axis_name="s",
                                num_cores=NC, num_subcores=NS)
  kern = pl.kernel(
      _sc_body,
      out_type=[
          jax.ShapeDtypeStruct((NC * NP, HALF), jnp.float32),
          jax.ShapeDtypeStruct((NC * NP, HALF), jnp.float32),
      ],
      mesh=mesh,
      scratch_types=[
          pltpu.VMEM((HC, CHUNK), jnp.int32),
          pltpu.VMEM((CH, CHUNK), jnp.int32),
          pltpu.VMEM((CHUNK, HALF), jnp.float32),
          pltpu.VMEM((CHUNK, HALF), jnp.float32),
          pltpu.VMEM_SHARED((NP, HALF), jnp.float32),
          pltpu.SemaphoreType.DMA,
          pltpu.SemaphoreType.DMA,
          pltpu.SemaphoreType.DMA,
          pltpu.SemaphoreType.DMA,
      ],
  )
  return kern(x, epk)


BN = 1000


def _tc_root_body(x_ref, wr_ref, bl_ref, o_ref):
  o_ref[...] = jnp.dot(x_ref[...], wr_ref[...],
                       preferred_element_type=jnp.float32) + bl_ref[...]


def _tc_root(x, wr_t, b_l):
  return pl.pallas_call(
      _tc_root_body,
      grid=(N // BN,),
      in_specs=[
          pl.BlockSpec((BN, D), lambda i: (i, 0)),
          pl.BlockSpec((D, D), lambda i: (0, 0)),
          pl.BlockSpec((1, D), lambda i: (0, 0)),
      ],
      out_specs=pl.BlockSpec((BN, D), lambda i: (i, 0)),
      out_shape=jax.ShapeDtypeStruct((N, D), jnp.float32),
  )(x, wr_t, b_l)


def _tc_body(hr_ref, a0_ref, a1_ref, c0_ref, c1_ref, wl_ref, wm_ref,
             bm_ref, o_ref):
  cnt = c0_ref[0][:, 0:1] + c1_ref[0][:, 0:1]
  inv = 1.0 / jnp.maximum(cnt, 1.0)
  a0 = a0_ref[0] * inv
  a1 = a1_ref[0] * inv
  h = (jnp.dot(a0, wl_ref[:HALF, :], preferred_element_type=jnp.float32)
       + jnp.dot(a1, wl_ref[HALF:, :], preferred_element_type=jnp.float32)
       + hr_ref[...])
  h = jnp.maximum(h, 0.0)
  o_ref[...] = jnp.dot(h, wm_ref[...],
                       preferred_element_type=jnp.float32) + bm_ref[...]


def _tc_head(hr, agg, cnt, wl_t, wm_t, b_mlp):
  grid = (N // BN,)
  return pl.pallas_call(
      _tc_body,
      grid=grid,
      in_specs=[
          pl.BlockSpec((BN, D), lambda i: (i, 0)),
          pl.BlockSpec((1, BN, HALF), lambda i: (0, i, 0)),
          pl.BlockSpec((1, BN, HALF), lambda i: (1, i, 0)),
          pl.BlockSpec((1, BN, HALF), lambda i: (0, i, 0)),
          pl.BlockSpec((1, BN, HALF), lambda i: (1, i, 0)),
          pl.BlockSpec((D, D), lambda i: (0, 0)),
          pl.BlockSpec((D, C), lambda i: (0, 0)),
          pl.BlockSpec((1, C), lambda i: (0, 0)),
      ],
      out_specs=pl.BlockSpec((BN, C), lambda i: (i, 0)),
      out_shape=jax.ShapeDtypeStruct((N, C), jnp.float32),
  )(hr, agg, agg, cnt, cnt, wl_t, wm_t, b_mlp)


def kernel(x, edge_index, W_l, b_l, W_r, W_mlp, b_mlp):
  src = edge_index[0]
  dst = edge_index[1]
  pad = EP - E
  ar = jnp.arange(pad, dtype=jnp.int32)
  srcp = jnp.concatenate([src, (ar * 37) % N])
  dstp = jnp.concatenate([dst, N + ar % (NP - N)])
  epk = (srcp | (dstp << 14)).reshape(NS * CH, CHUNK)

  hr = _tc_root(x, W_r.T, b_l.reshape(1, D))
  agg, cnt = _sc_aggregate(x, epk)
  agg = agg.reshape(NC, NP, HALF)
  cnt = cnt.reshape(NC, NP, HALF)
  return _tc_head(hr, agg, cnt, W_l.T, W_mlp.T, b_mlp.reshape(1, C))


# speedup vs baseline: 1.0214x; 1.0214x over previous
import functools

import jax
import jax.numpy as jnp
from jax import lax
from jax.experimental import pallas as pl
from jax.experimental.pallas import tpu as pltpu
from jax.experimental.pallas import tpu_sc as plsc

N, E, D, C = 10000, 160000, 256, 64
HALF = D // 2
NC, NS = 2, 16
CHUNK = 128
HC = 40
CH = 2 * HC
EP = NS * CH * CHUNK
NP = 10240
STRIPE = NP // NS
LANES = 16
MASK14 = (1 << 14) - 1


def _fill(buf, val):
  vec = jnp.full((LANES,), val, jnp.float32)

  @pl.loop(0, CHUNK)
  def _(r):
    @pl.loop(0, HALF, step=LANES)
    def _(j):
      buf[r, pl.ds(j, LANES)] = vec


def _sc_body(x_hbm, epk, agg_out, cnt_out,
             pkh, idx_dst, rows0, rows1, agg_sh, sem0, sem1, sem2, sem3):
  c = lax.axis_index("c")
  s = lax.axis_index("s")
  base = s * STRIPE
  obase = c * NP + base
  xcol = x_hbm.at[:, pl.ds(pl.multiple_of(c * HALF, HALF), HALF)]

  _fill(rows0, 0.0)
  for j in range(STRIPE // CHUNK):
    pltpu.sync_copy(rows0, agg_sh.at[pl.ds(base + j * CHUNK, CHUNK)])

  plsc.subcore_barrier()

  for h in range(CH // HC):
    pltpu.sync_copy(epk.at[pl.ds(s * CH + h * HC, HC)], pkh)

    @pl.loop(0, HC)
    def _(r):
      @pl.loop(0, CHUNK, step=LANES)
      def _(j):
        v = pkh[r, pl.ds(j, LANES)]
        idx_dst[h * HC + r, pl.ds(j, LANES)] = v >> 14
        pkh[r, pl.ds(j, LANES)] = v & MASK14

    pltpu.make_async_copy(xcol.at[pkh.at[0]], rows0, sem0).start()
    pltpu.make_async_copy(xcol.at[pkh.at[1]], rows1, sem1).start()

    @pl.loop(0, HC, step=2)
    def _(g):
      pltpu.make_async_copy(xcol.at[pkh.at[g]], rows0, sem0).wait()
      pltpu.sync_copy(rows0, agg_sh.at[idx_dst.at[h * HC + g]], add=True)

      @pl.when(g + 2 < HC)
      def _():
        pltpu.make_async_copy(xcol.at[pkh.at[g + 2]], rows0, sem0).start()

      pltpu.make_async_copy(xcol.at[pkh.at[g + 1]], rows1, sem1).wait()
      pltpu.sync_copy(rows1, agg_sh.at[idx_dst.at[h * HC + g + 1]],
                      add=True)

      @pl.when(g + 3 < HC)
      def _():
        pltpu.make_async_copy(xcol.at[pkh.at[g + 3]], rows1, sem1).start()

  plsc.subcore_barrier()

  pltpu.sync_copy(agg_sh.at[pl.ds(base, STRIPE)],
                  agg_out.at[pl.ds(obase, STRIPE)])

  plsc.subcore_barrier()

  _fill(rows1, 1.0)

  @pl.loop(0, CH, step=8)
  def _(g):
    for k in range(4):
      pltpu.make_async_copy(
          rows1, agg_sh.at[idx_dst.at[g + 2 * k + c]], sem2).start(add=True)
    for k in range(4):
      pltpu.make_async_copy(
          rows1, agg_sh.at[idx_dst.at[g + 2 * k + c]], sem2).wait()

  plsc.subcore_barrier()

  pltpu.sync_copy(agg_sh.at[pl.ds(base, STRIPE)],
                  cnt_out.at[pl.ds(obase, STRIPE)])


def _sc_aggregate(x, epk):
  mesh = plsc.VectorSubcoreMesh(core_axis_name="c", subcore_axis_name="s",
                                num_cores=NC, num_subcores=NS)
  kern = pl.kernel(
      _sc_body,
      out_type=[
          jax.ShapeDtypeStruct((NC * NP, HALF), jnp.float32),
          jax.ShapeDtypeStruct((NC * NP, HALF), jnp.float32),
      ],
      mesh=mesh,
      scratch_types=[
          pltpu.VMEM((HC, CHUNK), jnp.int32),
          pltpu.VMEM((CH, CHUNK), jnp.int32),
          pltpu.VMEM((CHUNK, HALF), jnp.float32),
          pltpu.VMEM((CHUNK, HALF), jnp.float32),
          pltpu.VMEM_SHARED((NP, HALF), jnp.float32),
          pltpu.SemaphoreType.DMA,
          pltpu.SemaphoreType.DMA,
          pltpu.SemaphoreType.DMA,
          pltpu.SemaphoreType.DMA,
      ],
  )
  return kern(x, epk)


BN = 1000


def _tc_root_body(x_ref, wr_ref, bl_ref, o_ref):
  o_ref[...] = jnp.dot(x_ref[...], wr_ref[...],
                       preferred_element_type=jnp.float32) + bl_ref[...]


def _tc_root(x, wr_t, b_l):
  return pl.pallas_call(
      _tc_root_body,
      grid=(N // BN,),
      in_specs=[
          pl.BlockSpec((BN, D), lambda i: (i, 0)),
          pl.BlockSpec((D, D), lambda i: (0, 0)),
          pl.BlockSpec((1, D), lambda i: (0, 0)),
      ],
      out_specs=pl.BlockSpec((BN, D), lambda i: (i, 0)),
      out_shape=jax.ShapeDtypeStruct((N, D), jnp.float32),
  )(x, wr_t, b_l)


def _tc_body(hr_ref, a0_ref, a1_ref, c0_ref, c1_ref, wl_ref, wm_ref,
             bm_ref, o_ref):
  a0 = a0_ref[0]
  a1 = a1_ref[0]
  cnt = (c0_ref[0][:, 0:1] - a0[:, 0:1]) + (c1_ref[0][:, 0:1] - a1[:, 0:1])
  inv = 1.0 / jnp.maximum(cnt, 1.0)
  a0 = a0 * inv
  a1 = a1 * inv
  h = (jnp.dot(a0, wl_ref[:HALF, :], preferred_element_type=jnp.float32)
       + jnp.dot(a1, wl_ref[HALF:, :], preferred_element_type=jnp.float32)
       + hr_ref[...])
  h = jnp.maximum(h, 0.0)
  o_ref[...] = jnp.dot(h, wm_ref[...],
                       preferred_element_type=jnp.float32) + bm_ref[...]


def _tc_head(hr, agg, cnt, wl_t, wm_t, b_mlp):
  grid = (N // BN,)
  return pl.pallas_call(
      _tc_body,
      grid=grid,
      in_specs=[
          pl.BlockSpec((BN, D), lambda i: (i, 0)),
          pl.BlockSpec((1, BN, HALF), lambda i: (0, i, 0)),
          pl.BlockSpec((1, BN, HALF), lambda i: (1, i, 0)),
          pl.BlockSpec((1, BN, HALF), lambda i: (0, i, 0)),
          pl.BlockSpec((1, BN, HALF), lambda i: (1, i, 0)),
          pl.BlockSpec((D, D), lambda i: (0, 0)),
          pl.BlockSpec((D, C), lambda i: (0, 0)),
          pl.BlockSpec((1, C), lambda i: (0, 0)),
      ],
      out_specs=pl.BlockSpec((BN, C), lambda i: (i, 0)),
      out_shape=jax.ShapeDtypeStruct((N, C), jnp.float32),
  )(hr, agg, agg, cnt, cnt, wl_t, wm_t, b_mlp)


def kernel(x, edge_index, W_l, b_l, W_r, W_mlp, b_mlp):
  src = edge_index[0]
  dst = edge_index[1]
  pad = EP - E
  ar = jnp.arange(pad, dtype=jnp.int32)
  srcp = jnp.concatenate([src, (ar * 37) % N])
  dstp = jnp.concatenate([dst, N + ar % (NP - N)])
  epk = (srcp | (dstp << 14)).reshape(NS * CH, CHUNK)

  hr = _tc_root(x, W_r.T, b_l.reshape(1, D))
  agg, cnt = _sc_aggregate(x, epk)
  agg = agg.reshape(NC, NP, HALF)
  cnt = cnt.reshape(NC, NP, HALF)
  return _tc_head(hr, agg, cnt, W_l.T, W_mlp.T, b_mlp.reshape(1, C))
